# baseline (speedup 1.0000x reference)
import numpy as np
import jax
import jax.numpy as jnp
from jax.experimental import pallas as pl

N = 64
D = 256
OFFSETS = [0] + list(range(1, 16)) + list(range(17, 32, 2)) + list(range(35, 64, 4))
K = len(OFFSETS)


def _build_masks():
    V = np.zeros((K, N), dtype=np.float32)
    for k, o in enumerate(OFFSETS):
        for p in range(N):
            if p + o >= N:
                continue
            if 15 < o <= 31 and p % 2 != 0:
                continue
            if o > 31 and p % 4 != 0:
                continue
            V[k, p] = 1.0
    E = np.zeros((K, N), dtype=np.float32)
    for k, o in enumerate(OFFSETS):
        E[k, o] = 1.0
    return V, E


def _shift(a, amt):
    if amt == 0:
        return a
    return jnp.concatenate([a[:, amt:], a[:, :amt]], axis=1)


def _pool_kernel(x_ref, w1_ref, w2_ref, w3_ref, b_ref, v_ref, e_ref, out_ref):
    x = x_ref[0]
    gs = [x]
    g = x
    for o in range(1, 16):
        g = jnp.maximum(g, _shift(x, o))
        gs.append(g)
    g1, g3 = gs[1], gs[3]
    for o in range(17, 32, 2):
        g = jnp.maximum(g, _shift(g1, o - 1))
        gs.append(g)
    for o in range(35, 64, 4):
        g = jnp.maximum(g, _shift(g3, o - 3))
        gs.append(g)
    G = jnp.concatenate([v[:, None, :] for v in gs], axis=1)

    w2 = w2_ref[...]
    t2 = jax.lax.dot_general(
        w2, G.reshape(D, K * N),
        (((1,), (0,)), ((), ())),
        preferred_element_type=jnp.float32,
    ).reshape(D, K, N)
    y1 = jax.lax.dot_general(
        w1_ref[...], x, (((1,), (0,)), ((), ())),
        preferred_element_type=jnp.float32)
    y3 = jax.lax.dot_general(
        w3_ref[...], x, (((1,), (0,)), ((), ())),
        preferred_element_type=jnp.float32)
    y3s = jnp.concatenate(
        [_shift(y3, o)[:, None, :] for o in OFFSETS], axis=1)
    z = t2 + y1[:, None, :] + y3s
    zm = z * v_ref[...][None, :, :]

    skew = jax.lax.dot_general(
        zm, e_ref[...], (((1,), (0,)), ((), ())),
        preferred_element_type=jnp.float32)

    cur = skew
    for m in range(6):
        amt = 1 << m
        rolled = jnp.concatenate([cur[:, :, -amt:], cur[:, :, :-amt]], axis=2)
        pbit = (jax.lax.broadcasted_iota(jnp.int32, (D, N, N), 1) >> m) & 1
        cur = jnp.where(pbit == 1, rolled, cur)
    out_ref[0] = cur + b_ref[...][:, :, None]


def kernel(x, W, b):
    B = x.shape[0]
    w2d = W[:, :, 0, 0]
    w1 = w2d[:, :D]
    w2 = w2d[:, D:2 * D]
    w3 = w2d[:, 2 * D:]
    v_np, e_np = _build_masks()
    v = jnp.asarray(v_np)
    e = jnp.asarray(e_np)
    bias = b[:, None]

    return pl.pallas_call(
        _pool_kernel,
        grid=(B,),
        in_specs=[
            pl.BlockSpec((1, D, N), lambda i: (i, 0, 0)),
            pl.BlockSpec((D, D), lambda i: (0, 0)),
            pl.BlockSpec((D, D), lambda i: (0, 0)),
            pl.BlockSpec((D, D), lambda i: (0, 0)),
            pl.BlockSpec((D, 1), lambda i: (0, 0)),
            pl.BlockSpec((K, N), lambda i: (0, 0)),
            pl.BlockSpec((K, N), lambda i: (0, 0)),
        ],
        out_specs=pl.BlockSpec((1, D, N, N), lambda i: (i, 0, 0, 0)),
        out_shape=jax.ShapeDtypeStruct((B, D, N, N), x.dtype),
    )(x, w1, w2, w3, bias, v, e)


# speedup vs baseline: 5.2062x; 5.2062x over previous
import numpy as np
import jax
import jax.numpy as jnp
from jax.experimental import pallas as pl

N = 64
D = 256
OFFSETS = [0] + list(range(1, 16)) + list(range(17, 32, 2)) + list(range(35, 64, 4))
NEG = -1e30


def _active_mask() -> np.ndarray:
    A = np.zeros((N, N), dtype=np.float32)
    for o in OFFSETS:
        for p in range(N - o):
            if 15 < o <= 31 and p % 2 != 0:
                continue
            if o > 31 and p % 4 != 0:
                continue
            A[p, p + o] = 1.0
    return A


def _build_consts():
    A = _active_mask()
    f = np.arange(N * N)
    p, q = f // N, f % N
    dq = q - p
    pen = np.where(dq[None, :] >= (1 << np.arange(6))[:, None], 0.0, NEG)
    pen = pen.astype(np.float32)
    a_flat = A.reshape(1, N * N)
    R = np.zeros((2 * N + 1, N * N), dtype=np.float32)
    for pp in range(N):
        for qq in range(N):
            if A[pp, qq]:
                R[pp, pp * N + qq] = 1.0
                R[N + qq, pp * N + qq] = 1.0
    R[2 * N, :] = 1.0
    return a_flat, pen, R


def _pool_kernel(x_ref, w1_ref, w2_ref, w3_ref, b_ref, a_ref, pen_ref, r_ref,
                 out_ref):
    x = x_ref[0]
    c = jnp.tile(x, (1, N))
    for m in range(6):
        amt = 1 << m
        shifted = jnp.concatenate([c[:, -amt:], c[:, :-amt]], axis=1)
        c = jnp.maximum(c, shifted + pen_ref[m][None, :])
    gmap = c * a_ref[...]

    t2 = jax.lax.dot_general(
        w2_ref[...], gmap, (((1,), (0,)), ((), ())),
        preferred_element_type=jnp.float32)
    y1 = jax.lax.dot_general(
        w1_ref[...], x, (((1,), (0,)), ((), ())),
        preferred_element_type=jnp.float32)
    y3 = jax.lax.dot_general(
        w3_ref[...], x, (((1,), (0,)), ((), ())),
        preferred_element_type=jnp.float32)
    ycat = jnp.concatenate([y1, y3, b_ref[...]], axis=1)
    yterm = jax.lax.dot_general(
        ycat, r_ref[...], (((1,), (0,)), ((), ())),
        preferred_element_type=jnp.float32)
    out_ref[0] = t2 + yterm


def kernel(x, W, b):
    B = x.shape[0]
    w2d = W[:, :, 0, 0]
    w1 = w2d[:, :D]
    w2 = w2d[:, D:2 * D]
    w3 = w2d[:, 2 * D:]
    a_np, pen_np, r_np = _build_consts()
    a = jnp.asarray(a_np)
    pen = jnp.asarray(pen_np)
    r = jnp.asarray(r_np)
    bias = b[:, None]

    out_flat = pl.pallas_call(
        _pool_kernel,
        grid=(B,),
        in_specs=[
            pl.BlockSpec((1, D, N), lambda i: (i, 0, 0)),
            pl.BlockSpec((D, D), lambda i: (0, 0)),
            pl.BlockSpec((D, D), lambda i: (0, 0)),
            pl.BlockSpec((D, D), lambda i: (0, 0)),
            pl.BlockSpec((D, 1), lambda i: (0, 0)),
            pl.BlockSpec((1, N * N), lambda i: (0, 0)),
            pl.BlockSpec((6, N * N), lambda i: (0, 0)),
            pl.BlockSpec((2 * N + 1, N * N), lambda i: (0, 0)),
        ],
        out_specs=pl.BlockSpec((1, D, N * N), lambda i: (i, 0, 0)),
        out_shape=jax.ShapeDtypeStruct((B, D, N * N), x.dtype),
    )(x, w1, w2, w3, bias, a, pen, r)
    return out_flat.reshape(B, D, N, N)
